# baseline (speedup 1.0000x reference)
import jax
import jax.numpy as jnp
from jax.experimental import pallas as pl
from jax.experimental.pallas import tpu as pltpu

_OUT_DIM = 2048
_BM = 512


def _mm_kernel(scale_ref, x_ref, w_ref, b_ref, o_ref):
    w = w_ref[...].astype(jnp.bfloat16)
    acc = jax.lax.dot_general(
        x_ref[...], w,
        dimension_numbers=(((1,), (1,)), ((), ())),
        preferred_element_type=jnp.float32)
    scale = scale_ref[0, 0]
    o_ref[...] = (acc * scale + b_ref[...].astype(jnp.float32)).astype(
        jnp.bfloat16)


def kernel(x, weight_fp8, scale_w, bias):
    b, s, d = x.shape
    m = b * s
    x2 = x.reshape(m, d)
    bias2 = bias.reshape(1, _OUT_DIM)
    scale = scale_w.astype(jnp.float32).reshape(1, 1)
    out = pl.pallas_call(
        _mm_kernel,
        grid=(m // _BM,),
        in_specs=[
            pl.BlockSpec(memory_space=pltpu.SMEM),
            pl.BlockSpec((_BM, d), lambda i: (i, 0)),
            pl.BlockSpec((_OUT_DIM, d), lambda i: (0, 0)),
            pl.BlockSpec((1, _OUT_DIM), lambda i: (0, 0)),
        ],
        out_specs=pl.BlockSpec((_BM, _OUT_DIM), lambda i: (i, 0)),
        out_shape=jax.ShapeDtypeStruct((m, _OUT_DIM), jnp.bfloat16),
        compiler_params=pltpu.CompilerParams(
            dimension_semantics=("parallel",),
        ),
    )(scale, x2, weight_fp8, bias2)
    return out.reshape(b, s, _OUT_DIM)


# speedup vs baseline: 1.0316x; 1.0316x over previous
import jax
import jax.numpy as jnp
from jax.experimental import pallas as pl
from jax.experimental.pallas import tpu as pltpu

_OUT_DIM = 2048
_BM = 1024
_CORES = 2


def _mm_kernel(scale_ref, x_ref, w_ref, b_ref, o_ref, wt_ref):
    @pl.when(pl.program_id(1) == 0)
    def _dequant():
        s = scale_ref[0, 0].astype(jnp.bfloat16)
        wt_ref[...] = w_ref[...].astype(jnp.bfloat16) * s

    acc = jax.lax.dot_general(
        x_ref[...], wt_ref[...],
        dimension_numbers=(((1,), (1,)), ((), ())),
        preferred_element_type=jnp.float32)
    o_ref[...] = acc.astype(jnp.bfloat16) + b_ref[...]


def kernel(x, weight_fp8, scale_w, bias):
    b, s, d = x.shape
    m = b * s
    x2 = x.reshape(m, d)
    bias2 = bias.reshape(1, _OUT_DIM)
    scale = scale_w.astype(jnp.float32).reshape(1, 1)
    blocks_per_core = m // _BM // _CORES
    out = pl.pallas_call(
        _mm_kernel,
        grid=(_CORES, blocks_per_core),
        in_specs=[
            pl.BlockSpec(memory_space=pltpu.SMEM),
            pl.BlockSpec((_BM, d), lambda i, j: (i * blocks_per_core + j, 0)),
            pl.BlockSpec((_OUT_DIM, d), lambda i, j: (0, 0)),
            pl.BlockSpec((1, _OUT_DIM), lambda i, j: (0, 0)),
        ],
        out_specs=pl.BlockSpec(
            (_BM, _OUT_DIM), lambda i, j: (i * blocks_per_core + j, 0)),
        out_shape=jax.ShapeDtypeStruct((m, _OUT_DIM), jnp.bfloat16),
        scratch_shapes=[pltpu.VMEM((_OUT_DIM, d), jnp.bfloat16)],
        compiler_params=pltpu.CompilerParams(
            dimension_semantics=("parallel", "arbitrary"),
        ),
    )(scale, x2, weight_fp8, bias2)
    return out.reshape(b, s, _OUT_DIM)


# speedup vs baseline: 1.0458x; 1.0137x over previous
import jax
import jax.numpy as jnp
from jax.experimental import pallas as pl
from jax.experimental.pallas import tpu as pltpu

_OUT_DIM = 2048
_BM = 1024


def _mm_kernel(scale_ref, x_ref, w_ref, b_ref, o_ref):
    w = w_ref[...].astype(jnp.bfloat16)
    acc = jax.lax.dot_general(
        x_ref[...], w,
        dimension_numbers=(((1,), (1,)), ((), ())),
        preferred_element_type=jnp.float32)
    scale = scale_ref[0, 0]
    o_ref[...] = (acc * scale).astype(jnp.bfloat16) + b_ref[...]


def kernel(x, weight_fp8, scale_w, bias):
    b, s, d = x.shape
    m = b * s
    x2 = x.reshape(m, d)
    bias2 = bias.reshape(1, _OUT_DIM)
    scale = scale_w.astype(jnp.float32).reshape(1, 1)
    out = pl.pallas_call(
        _mm_kernel,
        grid=(m // _BM,),
        in_specs=[
            pl.BlockSpec(memory_space=pltpu.SMEM),
            pl.BlockSpec((_BM, d), lambda i: (i, 0)),
            pl.BlockSpec((_OUT_DIM, d), lambda i: (0, 0)),
            pl.BlockSpec((1, _OUT_DIM), lambda i: (0, 0)),
        ],
        out_specs=pl.BlockSpec((_BM, _OUT_DIM), lambda i: (i, 0)),
        out_shape=jax.ShapeDtypeStruct((m, _OUT_DIM), jnp.bfloat16),
        compiler_params=pltpu.CompilerParams(
            dimension_semantics=("parallel",),
        ),
    )(scale, x2, weight_fp8, bias2)
    return out.reshape(b, s, _OUT_DIM)
